# baseline (speedup 1.0000x reference)
import functools

import jax
import jax.numpy as jnp
from jax import lax
from jax.experimental import pallas as pl
from jax.experimental.pallas import tpu as pltpu
from jax.experimental.pallas import tpu_sc as plsc

_B = 1024
_NIMG = 4 * _B
_LANES = 128
_NT = 32
_LPT = _B // _NT


def _net_body(xr, w1r, b1r, w2r, b2r, f1r, g1r, f2r, g2r, f3r, g3r, outr):
    X3 = xr[...].reshape(28, 28, _LANES)
    p1 = []
    for c in range(6):
        acc = jnp.zeros((24, 24, _LANES), jnp.float32) + b1r[0, c]
        for di in range(5):
            for dj in range(5):
                acc = acc + w1r[c, 5 * di + dj] * X3[di:di + 24, dj:dj + 24, :]
        t = acc.reshape(12, 2, 12, 2, _LANES)
        t = jnp.max(jnp.max(t, axis=3), axis=1)
        p1.append(jnp.maximum(t, 0.0))
    p2 = []
    for o in range(16):
        acc = jnp.zeros((8, 8, _LANES), jnp.float32) + b2r[0, o]
        for c in range(6):
            for di in range(5):
                for dj in range(5):
                    acc = acc + w2r[o, 25 * c + 5 * di + dj] * p1[c][di:di + 8, dj:dj + 8, :]
        t = acc.reshape(4, 2, 4, 2, _LANES)
        t = jnp.max(jnp.max(t, axis=3), axis=1)
        p2.append(jnp.maximum(t, 0.0))
    flat = jnp.concatenate([t.reshape(16, _LANES) for t in p2], axis=0)
    h = jnp.dot(f1r[...], flat, preferred_element_type=jnp.float32) + g1r[...].reshape(120, 1)
    h = jnp.maximum(h, 0.0)
    h = jnp.dot(f2r[...], h, preferred_element_type=jnp.float32) + g2r[...].reshape(84, 1)
    h = jnp.maximum(h, 0.0)
    lg = jnp.dot(f3r[...], h, preferred_element_type=jnp.float32) + g3r[...].reshape(10, 1)
    m = jnp.max(lg, axis=0, keepdims=True)
    e = jnp.exp(lg - m)
    outr[...] = e / jnp.sum(e, axis=0, keepdims=True)


def _run_net(X2, w1m, b1m, w2m, b2m, Wf1, bf1m, Wf2, bf2m, Wf3, bf3m):
    nblk = _NIMG // _LANES
    full = lambda shape: pl.BlockSpec(shape, lambda i: (0,) * len(shape))
    return pl.pallas_call(
        _net_body,
        grid=(nblk,),
        in_specs=[
            pl.BlockSpec((784, _LANES), lambda i: (0, i)),
            full((6, 25)), full((1, 6)),
            full((16, 150)), full((1, 16)),
            full((120, 256)), full((1, 120)),
            full((84, 120)), full((1, 84)),
            full((10, 84)), full((1, 10)),
        ],
        out_specs=pl.BlockSpec((10, _LANES), lambda i: (0, i)),
        out_shape=jax.ShapeDtypeStruct((10, _NIMG), jnp.float32),
    )(X2, w1m, b1m, w2m, b2m, Wf1, bf1m, Wf2, bf2m, Wf3, bf3m)


def _sc_body(probs_hbm, out_hbm, pv, q1, q2r, acc):
    wid = lax.axis_index("s") * 2 + lax.axis_index("c")
    pltpu.sync_copy(probs_hbm.at[wid], pv)
    for g in range(2):
        sl = pl.ds(g * 16, 16)
        for a in range(10):
            va1 = pv[0, a, sl]
            va3 = pv[2, a, sl]
            for b in range(10):
                q1[10 * a + b, sl] = va1 * pv[1, b, sl]
                q2r[10 * (9 - a) + (9 - b), sl] = va3 * pv[3, b, sl]
        for r in range(24):
            acc[r, sl] = jnp.zeros((16,), jnp.float32)

        zero16 = jnp.zeros((16,), jnp.float32)

        def vbody(v, _):
            lo = jnp.maximum(0, v - 99)
            hi = jnp.minimum(99, v)
            off = 99 - v

            def ibody(i, M):
                x = q1[i, sl] * q2r[off + i, sl]
                out = []
                for r in range(8):
                    mr = M[r]
                    out.append(jnp.maximum(mr, x))
                    x = jnp.minimum(mr, x)
                return tuple(out)

            M = lax.fori_loop(lo, hi + 1, ibody, (zero16,) * 8)
            s = M[0]
            for r in range(1, 8):
                s = s + M[r]
            r0 = lax.rem(v, 10)
            r1 = lax.rem(lax.div(v, 10), 10)
            r2 = 20 + lax.div(v, 100)
            acc[r0, sl] += s
            acc[10 + r1, sl] += s
            acc[r2, sl] += s
            return 0

        lax.fori_loop(0, 199, vbody, 0)
    pltpu.sync_copy(acc, out_hbm.at[wid])


def _run_sc(probs_b):
    mesh = plsc.VectorSubcoreMesh(core_axis_name="c", subcore_axis_name="s")
    return pl.kernel(
        _sc_body,
        out_type=jax.ShapeDtypeStruct((_NT, 24, _LPT), jnp.float32),
        mesh=mesh,
        scratch_types=[
            pltpu.VMEM((4, 10, _LPT), jnp.float32),
            pltpu.VMEM((100, _LPT), jnp.float32),
            pltpu.VMEM((100, _LPT), jnp.float32),
            pltpu.VMEM((24, _LPT), jnp.float32),
        ],
    )(probs_b)


def kernel(x, W1, b1, W2, b2, Wf1, bf1, Wf2, bf2, Wf3, bf3):
    B = x.shape[0]
    X2 = x.reshape(B, 4, 784).transpose(2, 1, 0).reshape(784, 4 * B)
    probs_dm = _run_net(
        X2,
        W1.reshape(6, 25), b1.reshape(1, 6),
        W2.reshape(16, 150), b2.reshape(1, 16),
        Wf1, bf1.reshape(1, 120),
        Wf2, bf2.reshape(1, 84),
        Wf3, bf3.reshape(1, 10),
    )
    probs_b = probs_dm.reshape(10, 4, _NT, _LPT).transpose(2, 1, 0, 3)
    out = _run_sc(probs_b)
    t0 = out[:, 0:10, :].transpose(0, 2, 1).reshape(B, 10)
    t1 = out[:, 10:20, :].transpose(0, 2, 1).reshape(B, 10)
    t2 = out[:, 20:22, :].transpose(0, 2, 1).reshape(B, 2)
    return (t0, t1, t2)


# speedup vs baseline: 1.2383x; 1.2383x over previous
import functools

import numpy as np
import jax
import jax.numpy as jnp
from jax import lax
from jax.experimental import pallas as pl
from jax.experimental.pallas import tpu as pltpu
from jax.experimental.pallas import tpu_sc as plsc

_B = 1024
_NIMG = 4 * _B
_LANES = 128
_NT = 32
_LPT = _B // _NT


def _conv_mat_indices(cout, cin, hin, win, k):
    hout, wout = hin - k + 1, win - k + 1
    o, i, j, c, di, dj = np.meshgrid(
        np.arange(cout), np.arange(hout), np.arange(wout),
        np.arange(cin), np.arange(k), np.arange(k), indexing="ij")
    rows = (o * hout + i) * wout + j
    cols = (c * hin + (i + di)) * win + (j + dj)
    widx = ((o * cin + c) * k + di) * k + dj
    return rows.ravel(), cols.ravel(), widx.ravel()

_R1, _C1, _W1I = _conv_mat_indices(6, 1, 28, 28, 5)
_R2, _C2, _W2I = _conv_mat_indices(16, 6, 12, 12, 5)


def _aug(mat, bias, pad_to):
    n, k = mat.shape
    return jnp.concatenate(
        [mat, bias.reshape(n, 1) if bias.ndim == 1 else bias,
         jnp.zeros((n, pad_to - k - 1), mat.dtype)], axis=1)


def _ones_pad(v, pad_to):
    k, L = v.shape
    return jnp.concatenate(
        [v, jnp.ones((1, L), v.dtype), jnp.zeros((pad_to - k - 1, L), v.dtype)],
        axis=0)


def _pool_relu(h, c, s):
    t = h.reshape(c, 2 * s, s, 2, _LANES)
    t = jnp.max(t, axis=3)
    t = t.reshape(c, s, 2, s, _LANES)
    t = jnp.max(t, axis=2)
    return jnp.maximum(t, 0.0).reshape(c * s * s, _LANES)


def _net_body(xr, m1r, m2r, f1r, f2r, f3r, outr):
    Xa = _ones_pad(xr[...], 792)
    h1 = jnp.dot(m1r[...], Xa, preferred_element_type=jnp.float32)
    p1 = _pool_relu(h1, 6, 12)
    h2 = jnp.dot(m2r[...], _ones_pad(p1, 872),
                 preferred_element_type=jnp.float32)
    p2 = _pool_relu(h2, 16, 4)
    h = jnp.dot(f1r[...], _ones_pad(p2, 264),
                preferred_element_type=jnp.float32)
    h = jnp.maximum(h, 0.0)
    h = jnp.dot(f2r[...], _ones_pad(h, 128),
                preferred_element_type=jnp.float32)
    h = jnp.maximum(h, 0.0)
    lg = jnp.dot(f3r[...], _ones_pad(h, 88),
                 preferred_element_type=jnp.float32)
    m = jnp.max(lg, axis=0, keepdims=True)
    e = jnp.exp(lg - m)
    outr[...] = e / jnp.sum(e, axis=0, keepdims=True)


def _run_net(X2, W1, b1, W2, b2, Wf1, bf1, Wf2, bf2, Wf3, bf3):
    m1 = jnp.zeros((3456, 784), jnp.float32).at[_R1, _C1].set(W1.ravel()[_W1I])
    m1 = _aug(m1, jnp.repeat(b1, 576), 792)
    m2 = jnp.zeros((1024, 864), jnp.float32).at[_R2, _C2].set(W2.ravel()[_W2I])
    m2 = _aug(m2, jnp.repeat(b2, 64), 872)
    f1 = _aug(Wf1, bf1, 264)
    f2 = _aug(Wf2, bf2, 128)
    f3 = _aug(Wf3, bf3, 88)
    nblk = _NIMG // _LANES
    full = lambda shape: pl.BlockSpec(shape, lambda i: (0,) * len(shape))
    return pl.pallas_call(
        _net_body,
        grid=(nblk,),
        in_specs=[
            pl.BlockSpec((784, _LANES), lambda i: (0, i)),
            full((3456, 792)), full((1024, 872)),
            full((120, 264)), full((84, 128)), full((10, 88)),
        ],
        out_specs=pl.BlockSpec((10, _LANES), lambda i: (0, i)),
        out_shape=jax.ShapeDtypeStruct((10, _NIMG), jnp.float32),
    )(X2, m1, m2, f1, f2, f3)


def _sc_body(probs_hbm, out_hbm, pv, q1, q2r, acc):
    wid = lax.axis_index("s") * 2 + lax.axis_index("c")
    pltpu.sync_copy(probs_hbm.at[wid], pv)
    for g in range(2):
        sl = pl.ds(g * 16, 16)
        for a in range(10):
            va1 = pv[0, a, sl]
            va3 = pv[2, a, sl]
            for b in range(10):
                q1[10 * a + b, sl] = va1 * pv[1, b, sl]
                q2r[10 * (9 - a) + (9 - b), sl] = va3 * pv[3, b, sl]
        for r in range(24):
            acc[r, sl] = jnp.zeros((16,), jnp.float32)

        zero16 = jnp.zeros((16,), jnp.float32)

        def vbody(v, _):
            lo = jnp.maximum(0, v - 99)
            hi = jnp.minimum(99, v)
            off = 99 - v

            def ibody(i, M):
                x = q1[i, sl] * q2r[off + i, sl]
                out = []
                for r in range(8):
                    mr = M[r]
                    out.append(jnp.maximum(mr, x))
                    x = jnp.minimum(mr, x)
                return tuple(out)

            M = lax.fori_loop(lo, hi + 1, ibody, (zero16,) * 8)
            s = M[0]
            for r in range(1, 8):
                s = s + M[r]
            r0 = lax.rem(v, 10)
            r1 = lax.rem(lax.div(v, 10), 10)
            r2 = 20 + lax.div(v, 100)
            acc[r0, sl] += s
            acc[10 + r1, sl] += s
            acc[r2, sl] += s
            return 0

        lax.fori_loop(0, 199, vbody, 0)
    pltpu.sync_copy(acc, out_hbm.at[wid])


def _run_sc(probs_b):
    mesh = plsc.VectorSubcoreMesh(core_axis_name="c", subcore_axis_name="s")
    return pl.kernel(
        _sc_body,
        out_type=jax.ShapeDtypeStruct((_NT, 24, _LPT), jnp.float32),
        mesh=mesh,
        scratch_types=[
            pltpu.VMEM((4, 10, _LPT), jnp.float32),
            pltpu.VMEM((100, _LPT), jnp.float32),
            pltpu.VMEM((100, _LPT), jnp.float32),
            pltpu.VMEM((24, _LPT), jnp.float32),
        ],
    )(probs_b)


def kernel(x, W1, b1, W2, b2, Wf1, bf1, Wf2, bf2, Wf3, bf3):
    B = x.shape[0]
    X2 = x.reshape(B, 4, 784).transpose(2, 1, 0).reshape(784, 4 * B)
    probs_dm = _run_net(
        X2, W1, b1, W2, b2, Wf1, bf1, Wf2, bf2, Wf3, bf3,
    )
    probs_b = probs_dm.reshape(10, 4, _NT, _LPT).transpose(2, 1, 0, 3)
    out = _run_sc(probs_b)
    t0 = out[:, 0:10, :].transpose(0, 2, 1).reshape(B, 10)
    t1 = out[:, 10:20, :].transpose(0, 2, 1).reshape(B, 10)
    t2 = out[:, 20:22, :].transpose(0, 2, 1).reshape(B, 2)
    return (t0, t1, t2)
